# baseline (speedup 1.0000x reference)
import functools

import jax
import jax.numpy as jnp
from jax.experimental import pallas as pl

N_Q = 8
K = 1024
D = 128
ROWS = 12000
BLOCK_ROWS = 2400
N_CHUNKS = 12
CHUNK = BLOCK_ROWS // N_CHUNKS


def _rowsum128(x):
    acc = x[:, 0:8]
    for t in range(1, 16):
        acc = acc + x[:, 8 * t:8 * t + 8]
    a = acc[:, :4] + acc[:, 4:8]
    b = a[:, :2] + a[:, 2:4]
    return b[:, 0:1] + b[:, 1:2]


def _colsum128(x):
    acc = x[0:8]
    for t in range(1, 16):
        acc = acc + x[8 * t:8 * t + 8]
    a = acc[:4] + acc[4:8]
    b = a[:2] + a[2:4]
    return b[0:1] + b[1:2]


def _split3(cb):
    mask = jnp.uint32(0xFFFF0000)
    hi_f = jax.lax.bitcast_convert_type(
        jax.lax.bitcast_convert_type(cb, jnp.uint32) & mask, jnp.float32)
    r1 = cb - hi_f
    mid_f = jax.lax.bitcast_convert_type(
        jax.lax.bitcast_convert_type(r1, jnp.uint32) & mask, jnp.float32)
    return (hi_f.astype(jnp.bfloat16), mid_f.astype(jnp.bfloat16),
            (r1 - mid_f).astype(jnp.bfloat16))


def _rvq_body(z_ref, cb_ref, cbt_ref, out_ref, codes_ref):
    col = jax.lax.broadcasted_iota(jnp.int32, (CHUNK, K), 1)
    dn_t = (((1,), (1,)), ((), ()))
    dn = (((1,), (0,)), ((), ()))

    residual = [z_ref[c * CHUNK:(c + 1) * CHUNK, :] for c in range(N_CHUNKS)]
    quantized = [jnp.zeros((CHUNK, D), jnp.float32) for _ in range(N_CHUNKS)]
    codes_cols = [[] for _ in range(N_CHUNKS)]

    for q in range(N_Q):
        cb = cb_ref[q]
        cbt = cbt_ref[q]
        cb_bf = cb.astype(jnp.bfloat16)
        cb_hi, cb_mid, cb_lo = _split3(cb)
        c2 = _colsum128(cbt * cbt)
        for c in range(N_CHUNKS):
            res = residual[c]
            r2 = _rowsum128(res * res)
            rneg2 = res.astype(jnp.bfloat16) * jnp.bfloat16(-2.0)
            rc2 = jax.lax.dot_general(rneg2, cb_bf, dn_t,
                                      preferred_element_type=jnp.float32)
            dist = r2 + rc2 + c2
            m = jnp.min(dist, axis=1, keepdims=True)
            idx = jnp.min(jnp.where(dist == m, col, K), axis=1, keepdims=True)
            onehot = (col == idx).astype(jnp.bfloat16)
            sel = (jax.lax.dot_general(onehot, cb_hi, dn,
                                       preferred_element_type=jnp.float32)
                   + jax.lax.dot_general(onehot, cb_mid, dn,
                                         preferred_element_type=jnp.float32)
                   ) + jax.lax.dot_general(onehot, cb_lo, dn,
                                           preferred_element_type=jnp.float32)
            quantized[c] = quantized[c] + sel
            residual[c] = res - sel
            codes_cols[c].append(idx)

    for c in range(N_CHUNKS):
        lo, hi = c * CHUNK, (c + 1) * CHUNK
        flat = z_ref[lo:hi, :]
        out_ref[lo:hi, :] = flat + (quantized[c] - flat)
        codes_ref[lo:hi, :] = jnp.concatenate(codes_cols[c], axis=1)


@functools.partial(jax.jit, static_argnames=())
def kernel(z, codebooks):
    B, T, Dd = z.shape
    flat = z.reshape(ROWS, Dd)
    cbt = jnp.swapaxes(codebooks, 1, 2)
    grid = (ROWS // BLOCK_ROWS,)
    out, codes = pl.pallas_call(
        _rvq_body,
        grid=grid,
        in_specs=[
            pl.BlockSpec((BLOCK_ROWS, D), lambda i: (i, 0)),
            pl.BlockSpec((N_Q, K, D), lambda i: (0, 0, 0)),
            pl.BlockSpec((N_Q, D, K), lambda i: (0, 0, 0)),
        ],
        out_specs=[
            pl.BlockSpec((BLOCK_ROWS, D), lambda i: (i, 0)),
            pl.BlockSpec((BLOCK_ROWS, N_Q), lambda i: (i, 0)),
        ],
        out_shape=[
            jax.ShapeDtypeStruct((ROWS, D), jnp.float32),
            jax.ShapeDtypeStruct((ROWS, N_Q), jnp.int32),
        ],
    )(flat, codebooks, cbt)
    out = out.reshape(B, T, Dd)
    codes = codes.T.reshape(N_Q, B, T)
    return out, codes


# speedup vs baseline: 1.5861x; 1.5861x over previous
import functools

import jax
import jax.numpy as jnp
from jax.experimental import pallas as pl

N_Q = 8
K = 1024
D = 128
ROWS = 12000
BLOCK_ROWS = 4000
N_CHUNKS = 10
CHUNK = BLOCK_ROWS // N_CHUNKS


def _rowsum128(x):
    acc = x[:, 0:8]
    for t in range(1, 16):
        acc = acc + x[:, 8 * t:8 * t + 8]
    a = acc[:, :4] + acc[:, 4:8]
    b = a[:, :2] + a[:, 2:4]
    return b[:, 0:1] + b[:, 1:2]


def _colsum128(x):
    acc = x[0:8]
    for t in range(1, 16):
        acc = acc + x[8 * t:8 * t + 8]
    a = acc[:4] + acc[4:8]
    b = a[:2] + a[2:4]
    return b[0:1] + b[1:2]


def _split3(cb):
    mask = jnp.uint32(0xFFFF0000)
    hi_f = jax.lax.bitcast_convert_type(
        jax.lax.bitcast_convert_type(cb, jnp.uint32) & mask, jnp.float32)
    r1 = cb - hi_f
    mid_f = jax.lax.bitcast_convert_type(
        jax.lax.bitcast_convert_type(r1, jnp.uint32) & mask, jnp.float32)
    return (hi_f.astype(jnp.bfloat16), mid_f.astype(jnp.bfloat16),
            (r1 - mid_f).astype(jnp.bfloat16))


def _rvq_body(z_ref, cb_ref, cbt_ref, out_ref, codes_ref):
    col = jax.lax.broadcasted_iota(jnp.int32, (CHUNK, K), 1)
    dn_t = (((1,), (1,)), ((), ()))
    dn = (((1,), (0,)), ((), ()))

    residual = [z_ref[c * CHUNK:(c + 1) * CHUNK, :] for c in range(N_CHUNKS)]
    quantized = [jnp.zeros((CHUNK, D), jnp.float32) for _ in range(N_CHUNKS)]
    codes_cols = [[] for _ in range(N_CHUNKS)]

    for q in range(N_Q):
        cb = cb_ref[q]
        cbt = cbt_ref[q]
        cb_bf = cb.astype(jnp.bfloat16)
        cb_hi, cb_mid, cb_lo = _split3(cb)
        c2 = _colsum128(cbt * cbt)
        for c in range(N_CHUNKS):
            res = residual[c]
            r2 = _rowsum128(res * res)
            rneg2 = res.astype(jnp.bfloat16) * jnp.bfloat16(-2.0)
            rc2 = jax.lax.dot_general(rneg2, cb_bf, dn_t,
                                      preferred_element_type=jnp.float32)
            dist = r2 + rc2 + c2
            m = jnp.min(dist, axis=1, keepdims=True)
            idx = jnp.min(jnp.where(dist == m, col, K), axis=1, keepdims=True)
            onehot = (col == idx).astype(jnp.bfloat16)
            sel = (jax.lax.dot_general(onehot, cb_hi, dn,
                                       preferred_element_type=jnp.float32)
                   + jax.lax.dot_general(onehot, cb_mid, dn,
                                         preferred_element_type=jnp.float32)
                   ) + jax.lax.dot_general(onehot, cb_lo, dn,
                                           preferred_element_type=jnp.float32)
            quantized[c] = quantized[c] + sel
            residual[c] = res - sel
            codes_cols[c].append(idx)

    for c in range(N_CHUNKS):
        lo, hi = c * CHUNK, (c + 1) * CHUNK
        flat = z_ref[lo:hi, :]
        out_ref[lo:hi, :] = flat + (quantized[c] - flat)
        codes_ref[lo:hi, :] = jnp.concatenate(codes_cols[c], axis=1)


@functools.partial(jax.jit, static_argnames=())
def kernel(z, codebooks):
    B, T, Dd = z.shape
    flat = z.reshape(ROWS, Dd)
    cbt = jnp.swapaxes(codebooks, 1, 2)
    grid = (ROWS // BLOCK_ROWS,)
    out, codes = pl.pallas_call(
        _rvq_body,
        grid=grid,
        in_specs=[
            pl.BlockSpec((BLOCK_ROWS, D), lambda i: (i, 0)),
            pl.BlockSpec((N_Q, K, D), lambda i: (0, 0, 0)),
            pl.BlockSpec((N_Q, D, K), lambda i: (0, 0, 0)),
        ],
        out_specs=[
            pl.BlockSpec((BLOCK_ROWS, D), lambda i: (i, 0)),
            pl.BlockSpec((BLOCK_ROWS, N_Q), lambda i: (i, 0)),
        ],
        out_shape=[
            jax.ShapeDtypeStruct((ROWS, D), jnp.float32),
            jax.ShapeDtypeStruct((ROWS, N_Q), jnp.int32),
        ],
    )(flat, codebooks, cbt)
    out = out.reshape(B, T, Dd)
    codes = codes.T.reshape(N_Q, B, T)
    return out, codes
